# baseline (speedup 1.0000x reference)
import functools

import jax
import jax.numpy as jnp
from jax import lax
from jax.experimental import pallas as pl
from jax.experimental.pallas import tpu as pltpu

N = 10000
NP = 10240
E = 320000
K = 5000
CH = 16


def _tc_mm1(xp, W1r):
    def body(x_ref, w_ref, o_ref):
        o_ref[...] = jnp.dot(x_ref[...], w_ref[0],
                             preferred_element_type=jnp.float32)

    return pl.pallas_call(
        body,
        grid=(2, 40),
        in_specs=[
            pl.BlockSpec((256, 128), lambda c, m: (m, 0)),
            pl.BlockSpec((1, 128, 128), lambda c, m: (c, 0, 0)),
        ],
        out_specs=pl.BlockSpec((256, 128), lambda c, m: (c * 40 + m, 0)),
        out_shape=jax.ShapeDtypeStruct((2 * NP, 128), jnp.float32),
    )(xp, W1r)


def _tc_h(acc1, hlin3, sn3, b1r):
    def body(a_ref, h_ref, n_ref, b_ref, o_ref):
        hh = a_ref[0] + h_ref[0] * n_ref[0]
        o_ref[...] = jnp.maximum(hh + b_ref[0], 0.0)

    return pl.pallas_call(
        body,
        grid=(2, 40),
        in_specs=[
            pl.BlockSpec((1, 256, 128), lambda c, m: (c, m, 0)),
            pl.BlockSpec((1, 256, 128), lambda c, m: (c, m, 0)),
            pl.BlockSpec((1, 256, 1), lambda c, m: (0, m, 0)),
            pl.BlockSpec((1, 1, 128), lambda c, m: (c, 0, 0)),
        ],
        out_specs=pl.BlockSpec((256, 128), lambda c, m: (c * 40 + m, 0)),
        out_shape=jax.ShapeDtypeStruct((2 * NP, 128), jnp.float32),
    )(acc1, hlin3, sn3, b1r)


def _tc_score(agg3, h3, Wgrel, Wgroot, bgv):
    def body(a_ref, h_ref, wr_ref, wo_ref, bg_ref, s_ref, k_ref):
        m = pl.program_id(0)
        a = jnp.concatenate([a_ref[0], a_ref[1]], axis=1)
        h = jnp.concatenate([h_ref[0], h_ref[1]], axis=1)
        pre = (jnp.dot(a, wr_ref[...], preferred_element_type=jnp.float32)
               + jnp.dot(h, wo_ref[...], preferred_element_type=jnp.float32)
               + bg_ref[0, 0])
        sc = jnp.tanh(pre)
        sc = jnp.where(sc == 0.0, 0.0, sc)
        fi = m * 1024 + lax.broadcasted_iota(jnp.int32, (1024, 1), 0)
        sc = jnp.where(fi < N, sc, -jnp.inf)
        s_ref[...] = sc
        bits = lax.bitcast_convert_type(sc, jnp.int32)
        k_ref[...] = jnp.where(bits < 0, bits ^ jnp.int32(0x7FFFFFFF), bits)

    return pl.pallas_call(
        body,
        grid=(10,),
        in_specs=[
            pl.BlockSpec((2, 1024, 128), lambda m: (0, m, 0)),
            pl.BlockSpec((2, 1024, 128), lambda m: (0, m, 0)),
            pl.BlockSpec((256, 1), lambda m: (0, 0)),
            pl.BlockSpec((256, 1), lambda m: (0, 0)),
            pl.BlockSpec((1, 1), lambda m: (0, 0)),
        ],
        out_specs=[
            pl.BlockSpec((1024, 1), lambda m: (m, 0)),
            pl.BlockSpec((1024, 1), lambda m: (m, 0)),
        ],
        out_shape=[
            jax.ShapeDtypeStruct((NP, 1), jnp.float32),
            jax.ShapeDtypeStruct((NP, 1), jnp.int32),
        ],
    )(agg3, h3, Wgrel, Wgroot, bgv)


def _tc_rank(kcol, krow):
    def body(kc_ref, kr_ref, o_ref):
        m = pl.program_id(0)
        ki = kc_ref[...]
        fi = m * 1024 + lax.broadcasted_iota(jnp.int32, (1024, 1), 0)
        acc = jnp.zeros((1024, 128), jnp.int32)
        for jr in range(80):
            kj = kr_ref[jr, :].reshape(1, 128)
            fj = jr * 128 + lax.broadcasted_iota(jnp.int32, (1, 128), 1)
            gt = kj > ki
            tie = (kj == ki) & (fj < fi)
            acc = acc + (gt | tie).astype(jnp.int32)
        o_ref[...] = jnp.sum(acc, axis=1, keepdims=True)

    return pl.pallas_call(
        body,
        grid=(10,),
        in_specs=[
            pl.BlockSpec((1024, 1), lambda m: (m, 0)),
            pl.BlockSpec((80, 128), lambda m: (0, 0)),
        ],
        out_specs=pl.BlockSpec((1024, 1), lambda m: (m, 0)),
        out_shape=jax.ShapeDtypeStruct((NP, 1), jnp.int32),
    )(kcol, krow)


def _ordered_scatter(table, src, dst, scale):
    D = table.shape[1]
    s2 = src.reshape(-1, CH)
    d2 = dst.reshape(-1, CH)
    w2 = scale.reshape(-1, CH) if scale is not None else None

    def body(acc, xs):
        if w2 is None:
            s, d = xs
            for j in range(CH):
                acc = acc.at[d[j]].add(table[s[j]])
        else:
            s, d, w = xs
            for j in range(CH):
                acc = acc.at[d[j]].add(table[s[j]] * w[j])
        return acc, None

    xs = (s2, d2) if w2 is None else (s2, d2, w2)
    acc, _ = lax.scan(body, jnp.zeros((N, D), jnp.float32), xs)
    return acc


def kernel(x, edge_index, W1, b1, W2, b2, Wg_rel, Wg_root, bg):
    f32 = jnp.float32
    src = edge_index[0].astype(jnp.int32)
    dst = edge_index[1].astype(jnp.int32)
    xp = jnp.concatenate([x, jnp.zeros((NP - N, 128), f32)], axis=0)
    W1r = jnp.moveaxis(W1.reshape(128, 2, 128), 1, 0)
    b1r = b1.reshape(2, 1, 128)
    bgv = bg.reshape(1, 1)

    deg = jnp.zeros((N,), f32).at[dst].add(1.0) + 1.0
    dinv = jnp.where(deg > 0, 1.0 / jnp.sqrt(jnp.where(deg > 0, deg, 1.0)), 0.0)
    norm = dinv[src] * dinv[dst]

    hlin = _tc_mm1(xp, W1r)
    hlin_n = jnp.concatenate([hlin[:N], hlin[NP:NP + N]], axis=1)
    acc1 = _ordered_scatter(hlin_n, src, dst, norm)
    acc1p = jnp.concatenate([acc1, jnp.zeros((NP - N, 256), f32)], 0)
    acc13 = jnp.moveaxis(acc1p.reshape(NP, 2, 128), 1, 0)
    sn3 = jnp.pad(dinv * dinv, (0, NP - N)).reshape(1, NP, 1)
    h = _tc_h(acc13, hlin.reshape(2, NP, 128), sn3, b1r)
    h3 = h.reshape(2, NP, 128)
    h_n = jnp.concatenate([h[:N], h[NP:NP + N]], axis=1)
    agg = _ordered_scatter(h_n, src, dst, None)
    aggp = jnp.concatenate([agg, jnp.zeros((NP - N, 256), f32)], 0)
    agg3 = jnp.moveaxis(aggp.reshape(NP, 2, 128), 1, 0)
    score_col, key_col = _tc_score(agg3, h3, Wg_rel, Wg_root, bgv)
    rank_col = _tc_rank(key_col, key_col.reshape(80, 128))
    rank = rank_col[:N, 0]
    S = score_col[:N, 0]
    keep = rank < K
    emask = keep[src] & keep[dst]
    s2 = jnp.where(emask, rank[src], 0)
    d2 = jnp.where(emask, rank[dst], 0)
    ew2 = emask.astype(f32)
    deg2 = jnp.zeros((N,), f32).at[dst].add(ew2) + 1.0
    dinv2 = jnp.where(deg2 > 0,
                      1.0 / jnp.sqrt(jnp.where(deg2 > 0, deg2, 1.0)), 0.0)
    h2s = ((h_n * S[:, None]) @ W2) * (keep.astype(f32) * dinv2)[:, None]
    acc2 = jnp.zeros((N, 256), f32).at[dst].add(h2s[src])
    z = jax.nn.relu(dinv2[:, None] * (acc2 + h2s) + b2)
    rc = jnp.minimum(rank, K)
    out = jnp.zeros((K + 1, 256), f32).at[rc].set(z)[:K]
    return out, jnp.stack([s2, d2]), emask
